# baseline (device time: 254655 ns/iter reference)
import jax
import jax.numpy as jnp
from jax import lax
from jax.experimental import pallas as pl
from jax.experimental.pallas import tpu as pltpu

N_DEV = 32


def kernel(x, w_mat, scale_x, scale_w):
    m_per, k = x.shape
    n_per = w_mat.shape[1]

    x8 = x.astype(jnp.float8_e4m3fn)
    w16 = w_mat.astype(jnp.bfloat16)

    def body(x_ref, w_ref, sx_ref, sw_ref, out_ref, comm_ref, send_sems, recv_sems):
        my = lax.axis_index("i")
        left = lax.rem(my - 1 + N_DEV, N_DEV)
        right = lax.rem(my + 1, N_DEV)

        barrier_sem = pltpu.get_barrier_semaphore()
        pl.semaphore_signal(barrier_sem, inc=1, device_id=(left,),
                            device_id_type=pl.DeviceIdType.MESH)
        pl.semaphore_signal(barrier_sem, inc=1, device_id=(right,),
                            device_id_type=pl.DeviceIdType.MESH)
        pl.semaphore_wait(barrier_sem, 2)

        scale = sx_ref[0] * sw_ref[0]
        w = w_ref[...]

        def mm_store(origin, chunk8):
            acc = jnp.dot(chunk8.astype(jnp.bfloat16), w,
                          preferred_element_type=jnp.float32)
            out_ref[pl.ds(origin * m_per, m_per), :] = jnp.maximum(acc * scale, 0.0)

        comm_ref[0] = x_ref[...]
        mm_store(my, x_ref[...])

        for h in range(1, N_DEV):
            rdma = pltpu.make_async_remote_copy(
                src_ref=comm_ref.at[h - 1],
                dst_ref=comm_ref.at[h],
                send_sem=send_sems.at[h - 1],
                recv_sem=recv_sems.at[h],
                device_id=(right,),
                device_id_type=pl.DeviceIdType.MESH,
            )
            rdma.start()
            rdma.wait()
            origin = lax.rem(my - h + N_DEV, N_DEV)
            mm_store(origin, comm_ref[h])

    return pl.pallas_call(
        body,
        out_shape=jax.ShapeDtypeStruct((N_DEV * m_per, n_per), jnp.float32),
        in_specs=[
            pl.BlockSpec(memory_space=pltpu.VMEM),
            pl.BlockSpec(memory_space=pltpu.VMEM),
            pl.BlockSpec(memory_space=pltpu.SMEM),
            pl.BlockSpec(memory_space=pltpu.SMEM),
        ],
        out_specs=pl.BlockSpec(memory_space=pltpu.VMEM),
        scratch_shapes=[
            pltpu.VMEM((N_DEV, m_per, k), jnp.float8_e4m3fn),
            pltpu.SemaphoreType.DMA((N_DEV,)),
            pltpu.SemaphoreType.DMA((N_DEV,)),
        ],
        compiler_params=pltpu.CompilerParams(collective_id=0),
    )(x8, w16, scale_x, scale_w)


# device time: 189832 ns/iter; 1.3415x vs baseline; 1.3415x over previous
import jax
import jax.numpy as jnp
from jax import lax
from jax.experimental import pallas as pl
from jax.experimental.pallas import tpu as pltpu

N_DEV = 32
HR = N_DEV // 2
HL = N_DEV - 1 - HR


def kernel(x, w_mat, scale_x, scale_w):
    m_per, k = x.shape
    n_per = w_mat.shape[1]

    x8 = x.astype(jnp.float8_e4m3fn)
    w16 = w_mat.astype(jnp.bfloat16)

    def body(x_ref, w_ref, sx_ref, sw_ref, out_ref,
             buf_r, buf_l, send_r, recv_r, send_l, recv_l):
        my = lax.axis_index("i")
        left = lax.rem(my - 1 + N_DEV, N_DEV)
        right = lax.rem(my + 1, N_DEV)

        barrier_sem = pltpu.get_barrier_semaphore()
        pl.semaphore_signal(barrier_sem, inc=1, device_id=(left,),
                            device_id_type=pl.DeviceIdType.MESH)
        pl.semaphore_signal(barrier_sem, inc=1, device_id=(right,),
                            device_id_type=pl.DeviceIdType.MESH)
        pl.semaphore_wait(barrier_sem, 2)

        scale = sx_ref[0] * sw_ref[0]
        w = w_ref[...]

        def mm_store(origin, chunk8):
            acc = jnp.dot(chunk8.astype(jnp.bfloat16), w,
                          preferred_element_type=jnp.float32)
            out_ref[pl.ds(origin * m_per, m_per), :] = jnp.maximum(acc * scale, 0.0)

        desc_r = [
            pltpu.make_async_remote_copy(
                src_ref=(x_ref if h == 1 else buf_r.at[h - 2]),
                dst_ref=buf_r.at[h - 1],
                send_sem=send_r.at[h - 1],
                recv_sem=recv_r.at[h - 1],
                device_id=(right,),
                device_id_type=pl.DeviceIdType.MESH,
            )
            for h in range(1, HR + 1)
        ]
        desc_l = [
            pltpu.make_async_remote_copy(
                src_ref=(x_ref if h == 1 else buf_l.at[h - 2]),
                dst_ref=buf_l.at[h - 1],
                send_sem=send_l.at[h - 1],
                recv_sem=recv_l.at[h - 1],
                device_id=(left,),
                device_id_type=pl.DeviceIdType.MESH,
            )
            for h in range(1, HL + 1)
        ]

        desc_r[0].start()
        desc_l[0].start()
        mm_store(my, x_ref[...])

        for h in range(1, HR + 1):
            desc_r[h - 1].wait_recv()
            if h < HR:
                desc_r[h].start()
            if h <= HL:
                desc_l[h - 1].wait_recv()
                if h < HL:
                    desc_l[h].start()
            mm_store(lax.rem(my - h + N_DEV, N_DEV), buf_r[h - 1])
            if h <= HL:
                mm_store(lax.rem(my + h, N_DEV), buf_l[h - 1])

        for d in desc_r:
            d.wait_send()
        for d in desc_l:
            d.wait_send()

    return pl.pallas_call(
        body,
        out_shape=jax.ShapeDtypeStruct((N_DEV * m_per, n_per), jnp.float32),
        in_specs=[
            pl.BlockSpec(memory_space=pltpu.VMEM),
            pl.BlockSpec(memory_space=pltpu.VMEM),
            pl.BlockSpec(memory_space=pltpu.SMEM),
            pl.BlockSpec(memory_space=pltpu.SMEM),
        ],
        out_specs=pl.BlockSpec(memory_space=pltpu.VMEM),
        scratch_shapes=[
            pltpu.VMEM((HR, m_per, k), jnp.float8_e4m3fn),
            pltpu.VMEM((HL, m_per, k), jnp.float8_e4m3fn),
            pltpu.SemaphoreType.DMA((HR,)),
            pltpu.SemaphoreType.DMA((HR,)),
            pltpu.SemaphoreType.DMA((HL,)),
            pltpu.SemaphoreType.DMA((HL,)),
        ],
        compiler_params=pltpu.CompilerParams(collective_id=0),
    )(x8, w16, scale_x, scale_w)
